# baseline (speedup 1.0000x reference)
import functools

import numpy as np
import jax
import jax.numpy as jnp
from jax import lax
from jax.experimental import pallas as pl
from jax.experimental.pallas import tpu as pltpu
from jax.experimental.pallas import tpu_sc as plsc

_B = 16
_G = 32
_NC = 96

_ATAN_C = (1.0, -0.33333138, 0.19993694, -0.14211106, 0.10667487,
           -0.075569004, 0.043278243, -0.01641319, 0.002932762)
_LOG1P_C = (1.0, -0.49999905, 0.33330005, -0.24954559, 0.19678117,
            -0.15311863, 0.10614265, -0.0570642, 0.019907162, -0.0032563785)
_HALF_PI = float(np.pi / 2)
_V_COEF = float(4.0 / (np.pi ** 2))


def _poly(x, coeffs):
    acc = jnp.full((16,), coeffs[-1], jnp.float32)
    for c in coeffs[-2::-1]:
        acc = acc * x + c
    return acc


def _atan_pos(q):
    r = 1.0 / q
    z = jnp.minimum(q, r)
    a = z * _poly(z * z, _ATAN_C)
    return jnp.where(q > 1.0, _HALF_PI - a, a)


def _softplus16(l):
    y = jnp.exp(-jnp.abs(l))
    return jnp.maximum(l, 0.0) + y * _poly(y, _LOG1P_C)


def _sigmoid16(x):
    return 1.0 / (1.0 + jnp.exp(-x))


def _lane_total(x, brow):
    brow[...] = plsc.cumsum(x)
    return plsc.load_gather(brow, [jnp.full((16,), 15, jnp.int32)])


def _sc_body(tgt_hbm, p0, p1, p2, out_hbm,
             bmap, tgv, dstv, idxv, dbuf, accv, brow, sem):
    cid = lax.axis_index("c")
    sid = lax.axis_index("s")
    wid = sid * 2 + cid
    iota = lax.iota(jnp.int32, 16)
    zero16 = jnp.zeros((16,), jnp.float32)

    db = wid // 2
    dh = wid % 2
    cp0 = pltpu.async_copy(p0.at[pl.ds(db * 32000 + dh * 3200, 3200)],
                           dbuf.at[pl.ds(0, 3200)], sem)
    cp1 = pltpu.async_copy(p1.at[pl.ds(db * 8000 + dh * 800, 800)],
                           dbuf.at[pl.ds(3200, 800)], sem)
    cp0.wait()
    cp1.wait()

    @pl.when(wid < 16)
    def _():
        pltpu.async_copy(p2.at[pl.ds(wid * 2000, 400)],
                         dbuf.at[pl.ds(4000, 400)], sem).wait()

    niters = jnp.where(wid < 16, 275, 250)

    def dense_step(i, acc):
        return acc + _softplus16(dbuf[pl.ds(i * 16, 16)])

    soft_acc = lax.fori_loop(0, niters, dense_step, zero16)
    accv[...] = jnp.where(iota == 0, _lane_total(soft_acc, brow), 0.0)

    def run_task(task):
        level = task // 16
        b = task % 16
        W = jnp.where(level == 0, 80, jnp.where(level == 1, 40, 20)).astype(jnp.int32)
        HW = W * W
        Wf = jnp.where(level == 0, jnp.full((16,), 80.0, jnp.float32),
                       jnp.where(level == 1, jnp.full((16,), 40.0, jnp.float32),
                                 jnp.full((16,), 20.0, jnp.float32)))
        inv_s = jnp.where(level == 0, jnp.full((16,), 8.0 / 640.0, jnp.float32),
                          jnp.where(level == 1, jnp.full((16,), 16.0 / 640.0, jnp.float32),
                                    jnp.full((16,), 32.0 / 640.0, jnp.float32)))

        pltpu.sync_copy(tgt_hbm.at[b], tgv)

        cands = [None] * 6
        coords = [None] * 6
        tcomp = [None] * 2
        for h in range(2):
            gl = iota + (h * 16)
            cxg = plsc.load_gather(tgv, [gl, jnp.full((16,), 1, jnp.int32)])
            cyg = plsc.load_gather(tgv, [gl, jnp.full((16,), 2, jnp.int32)])
            twg = plsc.load_gather(tgv, [gl, jnp.full((16,), 3, jnp.int32)])
            thg = plsc.load_gather(tgv, [gl, jnp.full((16,), 4, jnp.int32)])
            tcomp[h] = (cxg, cyg, twg, thg)
            gx = cxg * Wf
            gy = cyg * Wf
            col = jnp.clip(gx.astype(jnp.int32), 0, W - 1)
            row = jnp.clip(gy.astype(jnp.int32), 0, W - 1)
            offx = gx - col.astype(jnp.float32)
            offy = gy - row.astype(jnp.float32)
            ltx = offx < 0.5
            lty = offy < 0.5
            nx = jnp.clip(jnp.where(ltx, col - 1, col + 1), 0, W - 1)
            vx = jnp.where(ltx, col > 0, col < W - 1)
            ny = jnp.clip(jnp.where(lty, row - 1, row + 1), 0, W - 1)
            vy = jnp.where(lty, row > 0, row < W - 1)
            neg = jnp.full((16,), -1, jnp.int32)
            cands[0 + h] = (row * W + col, gl)
            cands[2 + h] = (row * W + nx, jnp.where(vx, gl, neg))
            cands[4 + h] = (ny * W + col, jnp.where(vy, gl, neg))
            coords[0 + h] = (col, row)
            coords[2 + h] = (nx, row)
            coords[4 + h] = (col, ny)

        zi16 = jnp.zeros((16,), jnp.int32)
        one16 = jnp.ones((16,), jnp.int32)
        for idx, _ in cands:
            plsc.store_scatter(bmap, [idx], zi16)
        for idx, val in cands:
            vs = jnp.maximum(val, 0)
            plsc.addupdate_scatter(bmap, [idx], one16 << vs, mask=val >= 0)
        base = (b * 5) * HW
        wins = [None] * 6
        for k, (idx, val) in enumerate(cands):
            got = plsc.load_gather(bmap, [idx])
            vs = jnp.maximum(val, 0)
            above = jnp.full((16,), -2, jnp.int32) << vs
            wins[k] = ((val >= 0) & ((got & above) == 0)).astype(jnp.float32)
            cs = pl.ds(k * 16, 16)
            for c in range(5):
                idxv[c, cs] = idx + (base + c * HW)

        def gather_from(ph):
            def _go():
                hs = [pltpu.async_copy(ph.at[idxv.at[c]], dstv.at[c], sem)
                      for c in range(5)]
                for hh in hs:
                    hh.wait()
            return _go

        pl.when(level == 0)(gather_from(p0))
        pl.when(level == 1)(gather_from(p1))
        pl.when(level == 2)(gather_from(p2))

        eps = 1e-7
        npos_acc = zero16
        box_acc = zero16
        corr_acc = zero16
        at_t = [None] * 2
        for h in range(2):
            _, _, twg, thg = tcomp[h]
            at_t[h] = _atan_pos(twg / (thg + eps))
        for k in range(6):
            h = k % 2
            cs = pl.ds(k * 16, 16)
            lg = dstv[0, cs]
            q1 = dstv[1, cs]
            q2 = dstv[2, cs]
            q3 = dstv[3, cs]
            q4 = dstv[4, cs]
            colf = coords[k][0].astype(jnp.float32)
            rowf = coords[k][1].astype(jnp.float32)
            p_cx = (_sigmoid16(q1) * 2.0 - 0.5 + colf) * inv_s
            p_cy = (_sigmoid16(q2) * 2.0 - 0.5 + rowf) * inv_s
            p_w = jnp.exp(jnp.clip(q3, -5.0, 5.0)) * inv_s
            p_h = jnp.exp(jnp.clip(q4, -5.0, 5.0)) * inv_s
            t_cx, t_cy, t_w, t_h = tcomp[h]
            px1 = p_cx - p_w * 0.5
            py1 = p_cy - p_h * 0.5
            px2 = p_cx + p_w * 0.5
            py2 = p_cy + p_h * 0.5
            tx1 = t_cx - t_w * 0.5
            ty1 = t_cy - t_h * 0.5
            tx2 = t_cx + t_w * 0.5
            ty2 = t_cy + t_h * 0.5

            iw = jnp.maximum(jnp.minimum(px2, tx2) - jnp.maximum(px1, tx1), 0.0)
            ih = jnp.maximum(jnp.minimum(py2, ty2) - jnp.maximum(py1, ty1), 0.0)
            inter = iw * ih
            union = p_w * p_h + t_w * t_h - inter + eps
            iou = inter / union
            cw = jnp.maximum(px2, tx2) - jnp.minimum(px1, tx1)
            ch = jnp.maximum(py2, ty2) - jnp.minimum(py1, ty1)
            c2 = cw * cw + ch * ch + eps
            dx = tx1 + tx2 - px1 - px2
            dy = ty1 + ty2 - py1 - py2
            rho2 = (dx * dx + dy * dy) / 4.0
            dat = at_t[h] - _atan_pos(p_w / (p_h + eps))
            v = _V_COEF * (dat * dat)
            alpha = v / (v - iou + 1.0 + eps)
            cl = jnp.maximum(1.0 - (iou - (rho2 / c2 + alpha * v)), 0.0)

            ap = jnp.maximum(p_w * p_h, 1e-7)
            at = jnp.maximum(t_w * t_h, 1e-7)
            iou2 = jnp.clip(inter / (ap + at - inter + 1e-7), 0.0, 1.0)

            w = wins[k]
            npos_acc = npos_acc + w
            box_acc = box_acc + w * cl
            corr_acc = corr_acc + w * lg * iou2

        npos_t = _lane_total(npos_acc, brow)
        box_t = _lane_total(box_acc, brow)
        corr_t = _lane_total(corr_acc, brow)
        box_mean = jnp.where(npos_t > 0, box_t / jnp.maximum(npos_t, 1.0), 0.0)
        item = jnp.where(npos_t > 0, 1.0, 0.0)
        return (jnp.where(iota == 1, box_mean, 0.0)
                + jnp.where(iota == 2, item, 0.0)
                + jnp.where(iota == 3, npos_t, 0.0)
                + jnp.where(iota == 4, corr_t, 0.0))

    accv[...] = accv[...] + run_task(wid)

    @pl.when(wid < 16)
    def _():
        accv[...] = accv[...] + run_task(wid + 32)

    pltpu.sync_copy(accv, out_hbm.at[wid])


@jax.jit
def _sc_loss(tgt, p0f, p1f, p2f):
    mesh = plsc.VectorSubcoreMesh(core_axis_name="c", subcore_axis_name="s")
    return pl.kernel(
        _sc_body,
        out_type=jax.ShapeDtypeStruct((32, 16), jnp.float32),
        mesh=mesh,
        scratch_types=[
            pltpu.VMEM((6400,), jnp.int32),
            pltpu.VMEM((_G, 5), jnp.float32),
            pltpu.VMEM((5, _NC), jnp.float32),
            pltpu.VMEM((5, _NC), jnp.int32),
            pltpu.VMEM((4400,), jnp.float32),
            pltpu.VMEM((16,), jnp.float32),
            pltpu.VMEM((16,), jnp.float32),
            pltpu.SemaphoreType.DMA,
        ],
        compiler_params=pltpu.CompilerParams(needs_layout_passes=False),
    )(tgt, p0f, p1f, p2f)


def kernel(pred0, pred1, pred2, targets):
    p0f = pred0.reshape(-1)
    p1f = pred1.reshape(-1)
    p2f = pred2.reshape(-1)
    part = _sc_loss(targets, p0f, p1f, p2f)
    o = jnp.sum(part, axis=0)
    soft, box, items, tpos, corr = o[0], o[1], o[2], o[3], o[4]
    return (soft - corr) / jnp.maximum(1.0, tpos) + 5.0 * box / jnp.maximum(1.0, items)


# speedup vs baseline: 1.1067x; 1.1067x over previous
import functools

import numpy as np
import jax
import jax.numpy as jnp
from jax import lax
from jax.experimental import pallas as pl
from jax.experimental.pallas import tpu as pltpu
from jax.experimental.pallas import tpu_sc as plsc

_B = 16
_G = 32
_NC = 96
_BASE1 = 512000
_BASE2 = 640000

_ATAN_C = (1.0, -0.33333138, 0.19993694, -0.14211106, 0.10667487,
           -0.075569004, 0.043278243, -0.01641319, 0.002932762)
_LOG1P_C = (0.99999875, -0.4998719, 0.33112052, -0.23514864, 0.14943458,
            -0.06658805, 0.014202826)
_HALF_PI = float(np.pi / 2)
_V_COEF = float(4.0 / (np.pi ** 2))


def _poly(x, coeffs):
    acc = jnp.full((16,), coeffs[-1], jnp.float32)
    for c in coeffs[-2::-1]:
        acc = acc * x + c
    return acc


def _atan_pos(q):
    r = 1.0 / q
    z = jnp.minimum(q, r)
    a = z * _poly(z * z, _ATAN_C)
    return jnp.where(q > 1.0, _HALF_PI - a, a)


def _softplus16(l):
    y = jnp.exp(-jnp.abs(l))
    return jnp.maximum(l, 0.0) + y * _poly(y, _LOG1P_C)


def _sigmoid16(x):
    return 1.0 / (1.0 + jnp.exp(-x))


def _lane_total(x, brow):
    brow[...] = plsc.cumsum(x)
    return plsc.load_gather(brow, [jnp.full((16,), 15, jnp.int32)])


def _sc_body(tgt_hbm, pc, out_hbm,
             bmap, tgv, dstv, idxv, dbuf, accv, brow, semd, semg):
    cid = lax.axis_index("c")
    sid = lax.axis_index("s")
    wid = sid * 2 + cid
    iota = lax.iota(jnp.int32, 16)
    zero16 = jnp.zeros((16,), jnp.float32)

    db = wid // 2
    dh = wid % 2
    cp0 = pltpu.async_copy(pc.at[pl.ds(db * 32000 + dh * 3200, 3200)],
                           dbuf.at[pl.ds(0, 3200)], semd)
    cp1 = pltpu.async_copy(pc.at[pl.ds(_BASE1 + db * 8000 + dh * 800, 800)],
                           dbuf.at[pl.ds(3200, 800)], semd)

    @pl.when(wid < 16)
    def _():
        pltpu.async_copy(pc.at[pl.ds(_BASE2 + wid * 2000, 400)],
                         dbuf.at[pl.ds(4000, 400)], semd)

    def task_prep(task):
        level = task // 16
        b = task % 16
        W = jnp.where(level == 0, 80, jnp.where(level == 1, 40, 20)).astype(jnp.int32)
        HW = W * W
        Wf = jnp.where(level == 0, jnp.full((16,), 80.0, jnp.float32),
                       jnp.where(level == 1, jnp.full((16,), 40.0, jnp.float32),
                                 jnp.full((16,), 20.0, jnp.float32)))
        inv_s = jnp.where(level == 0, jnp.full((16,), 8.0 / 640.0, jnp.float32),
                          jnp.where(level == 1, jnp.full((16,), 16.0 / 640.0, jnp.float32),
                                    jnp.full((16,), 32.0 / 640.0, jnp.float32)))

        pltpu.sync_copy(tgt_hbm.at[b], tgv)

        cands = [None] * 6
        coords = [None] * 6
        tcomp = [None] * 2
        for h in range(2):
            gl = iota + (h * 16)
            cxg = plsc.load_gather(tgv, [gl, jnp.full((16,), 1, jnp.int32)])
            cyg = plsc.load_gather(tgv, [gl, jnp.full((16,), 2, jnp.int32)])
            twg = plsc.load_gather(tgv, [gl, jnp.full((16,), 3, jnp.int32)])
            thg = plsc.load_gather(tgv, [gl, jnp.full((16,), 4, jnp.int32)])
            tcomp[h] = (cxg, cyg, twg, thg)
            gx = cxg * Wf
            gy = cyg * Wf
            col = jnp.clip(gx.astype(jnp.int32), 0, W - 1)
            row = jnp.clip(gy.astype(jnp.int32), 0, W - 1)
            offx = gx - col.astype(jnp.float32)
            offy = gy - row.astype(jnp.float32)
            ltx = offx < 0.5
            lty = offy < 0.5
            nx = jnp.clip(jnp.where(ltx, col - 1, col + 1), 0, W - 1)
            vx = jnp.where(ltx, col > 0, col < W - 1)
            ny = jnp.clip(jnp.where(lty, row - 1, row + 1), 0, W - 1)
            vy = jnp.where(lty, row > 0, row < W - 1)
            neg = jnp.full((16,), -1, jnp.int32)
            cands[0 + h] = (row * W + col, gl)
            cands[2 + h] = (row * W + nx, jnp.where(vx, gl, neg))
            cands[4 + h] = (ny * W + col, jnp.where(vy, gl, neg))
            coords[0 + h] = (col, row)
            coords[2 + h] = (nx, row)
            coords[4 + h] = (col, ny)

        zi16 = jnp.zeros((16,), jnp.int32)
        one16 = jnp.ones((16,), jnp.int32)
        for idx, _ in cands:
            plsc.store_scatter(bmap, [idx], zi16)
        for idx, val in cands:
            vs = jnp.maximum(val, 0)
            plsc.addupdate_scatter(bmap, [idx], one16 << vs, mask=val >= 0)
        base = jnp.where(level == 0, 0, jnp.where(level == 1, _BASE1, _BASE2)) \
            + (b * 5) * HW
        wins = [None] * 6
        for k, (idx, val) in enumerate(cands):
            got = plsc.load_gather(bmap, [idx])
            vs = jnp.maximum(val, 0)
            above = jnp.full((16,), -2, jnp.int32) << vs
            wins[k] = ((val >= 0) & ((got & above) == 0)).astype(jnp.float32)
            cs = pl.ds(k * 16, 16)
            for c in range(5):
                idxv[c, cs] = idx + (base + c * HW)

        copies = [pltpu.async_copy(pc.at[idxv.at[c]], dstv.at[c], semg)
                  for c in range(5)]
        return (inv_s, coords, tcomp, wins, copies)

    def task_finish(state):
        inv_s, coords, tcomp, wins, copies = state
        for cp in copies:
            cp.wait()
        eps = 1e-7
        npos_acc = zero16
        box_acc = zero16
        corr_acc = zero16
        at_t = [None] * 2
        for h in range(2):
            _, _, twg, thg = tcomp[h]
            at_t[h] = _atan_pos(twg / (thg + eps))
        for k in range(6):
            h = k % 2
            cs = pl.ds(k * 16, 16)
            lg = dstv[0, cs]
            q1 = dstv[1, cs]
            q2 = dstv[2, cs]
            q3 = dstv[3, cs]
            q4 = dstv[4, cs]
            colf = coords[k][0].astype(jnp.float32)
            rowf = coords[k][1].astype(jnp.float32)
            p_cx = (_sigmoid16(q1) * 2.0 - 0.5 + colf) * inv_s
            p_cy = (_sigmoid16(q2) * 2.0 - 0.5 + rowf) * inv_s
            p_w = jnp.exp(jnp.clip(q3, -5.0, 5.0)) * inv_s
            p_h = jnp.exp(jnp.clip(q4, -5.0, 5.0)) * inv_s
            t_cx, t_cy, t_w, t_h = tcomp[h]
            px1 = p_cx - p_w * 0.5
            py1 = p_cy - p_h * 0.5
            px2 = p_cx + p_w * 0.5
            py2 = p_cy + p_h * 0.5
            tx1 = t_cx - t_w * 0.5
            ty1 = t_cy - t_h * 0.5
            tx2 = t_cx + t_w * 0.5
            ty2 = t_cy + t_h * 0.5

            iw = jnp.maximum(jnp.minimum(px2, tx2) - jnp.maximum(px1, tx1), 0.0)
            ih = jnp.maximum(jnp.minimum(py2, ty2) - jnp.maximum(py1, ty1), 0.0)
            inter = iw * ih
            union = p_w * p_h + t_w * t_h - inter + eps
            iou = inter / union
            cw = jnp.maximum(px2, tx2) - jnp.minimum(px1, tx1)
            ch = jnp.maximum(py2, ty2) - jnp.minimum(py1, ty1)
            c2 = cw * cw + ch * ch + eps
            dx = tx1 + tx2 - px1 - px2
            dy = ty1 + ty2 - py1 - py2
            rho2 = (dx * dx + dy * dy) / 4.0
            dat = at_t[h] - _atan_pos(p_w / (p_h + eps))
            v = _V_COEF * (dat * dat)
            alpha = v / (v - iou + 1.0 + eps)
            cl = jnp.maximum(1.0 - (iou - (rho2 / c2 + alpha * v)), 0.0)

            ap = jnp.maximum(p_w * p_h, 1e-7)
            at = jnp.maximum(t_w * t_h, 1e-7)
            iou2 = jnp.clip(inter / (ap + at - inter + 1e-7), 0.0, 1.0)

            w = wins[k]
            npos_acc = npos_acc + w
            box_acc = box_acc + w * cl
            corr_acc = corr_acc + w * lg * iou2

        npos_t = _lane_total(npos_acc, brow)
        box_t = _lane_total(box_acc, brow)
        corr_t = _lane_total(corr_acc, brow)
        box_mean = jnp.where(npos_t > 0, box_t / jnp.maximum(npos_t, 1.0), 0.0)
        item = jnp.where(npos_t > 0, 1.0, 0.0)
        return (jnp.where(iota == 1, box_mean, 0.0)
                + jnp.where(iota == 2, item, 0.0)
                + jnp.where(iota == 3, npos_t, 0.0)
                + jnp.where(iota == 4, corr_t, 0.0))

    state1 = task_prep(wid)

    cp0.wait()
    cp1.wait()

    @pl.when(wid < 16)
    def _():
        pltpu.make_async_copy(pc.at[pl.ds(_BASE2, 400)],
                              dbuf.at[pl.ds(4000, 400)], semd).wait()

    nouter = jnp.where(wid < 16, 55, 50)

    def dense_step(i, acc):
        for u in range(5):
            acc = acc + _softplus16(dbuf[pl.ds((i * 5 + u) * 16, 16)])
        return acc

    soft_acc = lax.fori_loop(0, nouter, dense_step, zero16)
    accv[...] = jnp.where(iota == 0, _lane_total(soft_acc, brow), 0.0)

    accv[...] = accv[...] + task_finish(state1)

    @pl.when(wid < 16)
    def _():
        accv[...] = accv[...] + task_finish(task_prep(wid + 32))

    pltpu.sync_copy(accv, out_hbm.at[wid])


@jax.jit
def _sc_loss(tgt, pcat):
    mesh = plsc.VectorSubcoreMesh(core_axis_name="c", subcore_axis_name="s")
    return pl.kernel(
        _sc_body,
        out_type=jax.ShapeDtypeStruct((32, 16), jnp.float32),
        mesh=mesh,
        scratch_types=[
            pltpu.VMEM((6400,), jnp.int32),
            pltpu.VMEM((_G, 5), jnp.float32),
            pltpu.VMEM((5, _NC), jnp.float32),
            pltpu.VMEM((5, _NC), jnp.int32),
            pltpu.VMEM((4400,), jnp.float32),
            pltpu.VMEM((16,), jnp.float32),
            pltpu.VMEM((16,), jnp.float32),
            pltpu.SemaphoreType.DMA,
            pltpu.SemaphoreType.DMA,
        ],
        compiler_params=pltpu.CompilerParams(needs_layout_passes=False),
    )(tgt, pcat)


def kernel(pred0, pred1, pred2, targets):
    pcat = jnp.concatenate(
        [pred0.reshape(-1), pred1.reshape(-1), pred2.reshape(-1)])
    part = _sc_loss(targets, pcat)
    o = jnp.sum(part, axis=0)
    soft, box, items, tpos, corr = o[0], o[1], o[2], o[3], o[4]
    return (soft - corr) / jnp.maximum(1.0, tpos) + 5.0 * box / jnp.maximum(1.0, items)


# speedup vs baseline: 1.1596x; 1.0478x over previous
import functools

import numpy as np
import jax
import jax.numpy as jnp
from jax import lax
from jax.experimental import pallas as pl
from jax.experimental.pallas import tpu as pltpu
from jax.experimental.pallas import tpu_sc as plsc

_B = 16
_G = 32
_NC = 96
_BASE1 = 512000
_BASE2 = 640000

_ATAN_C = (1.0, -0.33333138, 0.19993694, -0.14211106, 0.10667487,
           -0.075569004, 0.043278243, -0.01641319, 0.002932762)
_LOG1P_C = (0.99999875, -0.4998719, 0.33112052, -0.23514864, 0.14943458,
            -0.06658805, 0.014202826)
_HALF_PI = float(np.pi / 2)
_V_COEF = float(4.0 / (np.pi ** 2))


def _poly(x, coeffs):
    acc = jnp.full((16,), coeffs[-1], jnp.float32)
    for c in coeffs[-2::-1]:
        acc = acc * x + c
    return acc


def _atan_pos(q):
    r = 1.0 / q
    z = jnp.minimum(q, r)
    a = z * _poly(z * z, _ATAN_C)
    return jnp.where(q > 1.0, _HALF_PI - a, a)


def _softplus16(l):
    y = jnp.exp(-jnp.abs(l))
    return jnp.maximum(l, 0.0) + y * _poly(y, _LOG1P_C)


def _sigmoid16(x):
    return 1.0 / (1.0 + jnp.exp(-x))


def _lane_total(x, brow):
    brow[...] = plsc.cumsum(x)
    return plsc.load_gather(brow, [jnp.full((16,), 15, jnp.int32)])


def _sc_body(tgt_hbm, pc, out_hbm,
             bmap, tgv, dstv, idxv, dstv2, idxv2, dbuf, accv, brow,
             semd, semg, semg2, semt):
    cid = lax.axis_index("c")
    sid = lax.axis_index("s")
    wid = sid * 2 + cid
    iota = lax.iota(jnp.int32, 16)
    zero16 = jnp.zeros((16,), jnp.float32)

    cpt = pltpu.async_copy(tgt_hbm.at[wid % 16], tgv, semt)
    db = wid // 2
    dh = wid % 2
    cp0 = pltpu.async_copy(pc.at[pl.ds(db * 32000 + dh * 3200, 3200)],
                           dbuf.at[pl.ds(0, 3200)], semd)
    cp1 = pltpu.async_copy(pc.at[pl.ds(_BASE1 + db * 8000 + dh * 800, 800)],
                           dbuf.at[pl.ds(3200, 800)], semd)

    @pl.when(wid < 16)
    def _():
        pltpu.async_copy(pc.at[pl.ds(_BASE2 + wid * 2000, 400)],
                         dbuf.at[pl.ds(4000, 400)], semd)

    cpt.wait()

    def task_prep(task, idxv, dstv, semg):
        level = task // 16
        b = task % 16
        W = jnp.where(level == 0, 80, jnp.where(level == 1, 40, 20)).astype(jnp.int32)
        HW = W * W
        Wf = jnp.where(level == 0, jnp.full((16,), 80.0, jnp.float32),
                       jnp.where(level == 1, jnp.full((16,), 40.0, jnp.float32),
                                 jnp.full((16,), 20.0, jnp.float32)))
        inv_s = jnp.where(level == 0, jnp.full((16,), 8.0 / 640.0, jnp.float32),
                          jnp.where(level == 1, jnp.full((16,), 16.0 / 640.0, jnp.float32),
                                    jnp.full((16,), 32.0 / 640.0, jnp.float32)))

        cands = [None] * 6
        coords = [None] * 6
        tcomp = [None] * 2
        for h in range(2):
            gl = iota + (h * 16)
            cxg = plsc.load_gather(tgv, [gl, jnp.full((16,), 1, jnp.int32)])
            cyg = plsc.load_gather(tgv, [gl, jnp.full((16,), 2, jnp.int32)])
            twg = plsc.load_gather(tgv, [gl, jnp.full((16,), 3, jnp.int32)])
            thg = plsc.load_gather(tgv, [gl, jnp.full((16,), 4, jnp.int32)])
            tcomp[h] = (cxg, cyg, twg, thg)
            gx = cxg * Wf
            gy = cyg * Wf
            col = jnp.clip(gx.astype(jnp.int32), 0, W - 1)
            row = jnp.clip(gy.astype(jnp.int32), 0, W - 1)
            offx = gx - col.astype(jnp.float32)
            offy = gy - row.astype(jnp.float32)
            ltx = offx < 0.5
            lty = offy < 0.5
            nx = jnp.clip(jnp.where(ltx, col - 1, col + 1), 0, W - 1)
            vx = jnp.where(ltx, col > 0, col < W - 1)
            ny = jnp.clip(jnp.where(lty, row - 1, row + 1), 0, W - 1)
            vy = jnp.where(lty, row > 0, row < W - 1)
            neg = jnp.full((16,), -1, jnp.int32)
            cands[0 + h] = (row * W + col, gl)
            cands[2 + h] = (row * W + nx, jnp.where(vx, gl, neg))
            cands[4 + h] = (ny * W + col, jnp.where(vy, gl, neg))
            coords[0 + h] = (col, row)
            coords[2 + h] = (nx, row)
            coords[4 + h] = (col, ny)

        zi16 = jnp.zeros((16,), jnp.int32)
        one16 = jnp.ones((16,), jnp.int32)
        for idx, _ in cands:
            plsc.store_scatter(bmap, [idx], zi16)
        for idx, val in cands:
            vs = jnp.maximum(val, 0)
            plsc.addupdate_scatter(bmap, [idx], one16 << vs, mask=val >= 0)
        base = jnp.where(level == 0, 0, jnp.where(level == 1, _BASE1, _BASE2)) \
            + (b * 5) * HW
        wins = [None] * 6
        for k, (idx, val) in enumerate(cands):
            got = plsc.load_gather(bmap, [idx])
            vs = jnp.maximum(val, 0)
            above = jnp.full((16,), -2, jnp.int32) << vs
            wins[k] = ((val >= 0) & ((got & above) == 0)).astype(jnp.float32)
            cs = pl.ds(k * 16, 16)
            for c in range(5):
                idxv[c, cs] = idx + (base + c * HW)

        copies = [pltpu.async_copy(pc.at[idxv.at[c]], dstv.at[c], semg)
                  for c in range(5)]
        return (inv_s, coords, tcomp, wins, copies, dstv)

    def task_finish(state):
        inv_s, coords, tcomp, wins, copies, dstv = state
        for cp in copies:
            cp.wait()
        eps = 1e-7
        npos_acc = zero16
        box_acc = zero16
        corr_acc = zero16
        at_t = [None] * 2
        for h in range(2):
            _, _, twg, thg = tcomp[h]
            at_t[h] = _atan_pos(twg / (thg + eps))
        for k in range(6):
            h = k % 2
            cs = pl.ds(k * 16, 16)
            lg = dstv[0, cs]
            q1 = dstv[1, cs]
            q2 = dstv[2, cs]
            q3 = dstv[3, cs]
            q4 = dstv[4, cs]
            colf = coords[k][0].astype(jnp.float32)
            rowf = coords[k][1].astype(jnp.float32)
            p_cx = (_sigmoid16(q1) * 2.0 - 0.5 + colf) * inv_s
            p_cy = (_sigmoid16(q2) * 2.0 - 0.5 + rowf) * inv_s
            p_w = jnp.exp(jnp.clip(q3, -5.0, 5.0)) * inv_s
            p_h = jnp.exp(jnp.clip(q4, -5.0, 5.0)) * inv_s
            t_cx, t_cy, t_w, t_h = tcomp[h]
            px1 = p_cx - p_w * 0.5
            py1 = p_cy - p_h * 0.5
            px2 = p_cx + p_w * 0.5
            py2 = p_cy + p_h * 0.5
            tx1 = t_cx - t_w * 0.5
            ty1 = t_cy - t_h * 0.5
            tx2 = t_cx + t_w * 0.5
            ty2 = t_cy + t_h * 0.5

            iw = jnp.maximum(jnp.minimum(px2, tx2) - jnp.maximum(px1, tx1), 0.0)
            ih = jnp.maximum(jnp.minimum(py2, ty2) - jnp.maximum(py1, ty1), 0.0)
            inter = iw * ih
            union = p_w * p_h + t_w * t_h - inter + eps
            iou = inter / union
            cw = jnp.maximum(px2, tx2) - jnp.minimum(px1, tx1)
            ch = jnp.maximum(py2, ty2) - jnp.minimum(py1, ty1)
            c2 = cw * cw + ch * ch + eps
            dx = tx1 + tx2 - px1 - px2
            dy = ty1 + ty2 - py1 - py2
            rho2 = (dx * dx + dy * dy) / 4.0
            dat = at_t[h] - _atan_pos(p_w / (p_h + eps))
            v = _V_COEF * (dat * dat)
            alpha = v / (v - iou + 1.0 + eps)
            cl = jnp.maximum(1.0 - (iou - (rho2 / c2 + alpha * v)), 0.0)

            ap = jnp.maximum(p_w * p_h, 1e-7)
            at = jnp.maximum(t_w * t_h, 1e-7)
            iou2 = jnp.clip(inter / (ap + at - inter + 1e-7), 0.0, 1.0)

            w = wins[k]
            npos_acc = npos_acc + w
            box_acc = box_acc + w * cl
            corr_acc = corr_acc + w * lg * iou2

        npos_t = _lane_total(npos_acc, brow)
        box_t = _lane_total(box_acc, brow)
        corr_t = _lane_total(corr_acc, brow)
        box_mean = jnp.where(npos_t > 0, box_t / jnp.maximum(npos_t, 1.0), 0.0)
        item = jnp.where(npos_t > 0, 1.0, 0.0)
        return (jnp.where(iota == 1, box_mean, 0.0)
                + jnp.where(iota == 2, item, 0.0)
                + jnp.where(iota == 3, npos_t, 0.0)
                + jnp.where(iota == 4, corr_t, 0.0))

    state1 = task_prep(wid, idxv, dstv, semg)
    state2 = task_prep(wid + 32, idxv2, dstv2, semg2)

    cp0.wait()
    cp1.wait()

    @pl.when(wid < 16)
    def _():
        pltpu.make_async_copy(pc.at[pl.ds(_BASE2, 400)],
                              dbuf.at[pl.ds(4000, 400)], semd).wait()

    nouter = jnp.where(wid < 16, 55, 50)

    def dense_step(i, acc):
        for u in range(5):
            acc = acc + _softplus16(dbuf[pl.ds((i * 5 + u) * 16, 16)])
        return acc

    soft_acc = lax.fori_loop(0, nouter, dense_step, zero16)
    accv[...] = jnp.where(iota == 0, _lane_total(soft_acc, brow), 0.0)

    accv[...] = accv[...] + task_finish(state1)
    mask2 = jnp.where(wid < 16, jnp.full((16,), 1.0, jnp.float32), zero16)
    accv[...] = accv[...] + task_finish(state2) * mask2

    pltpu.sync_copy(accv, out_hbm.at[wid])


@jax.jit
def _sc_loss(tgt, pcat):
    mesh = plsc.VectorSubcoreMesh(core_axis_name="c", subcore_axis_name="s")
    return pl.kernel(
        _sc_body,
        out_type=jax.ShapeDtypeStruct((32, 16), jnp.float32),
        mesh=mesh,
        scratch_types=[
            pltpu.VMEM((6400,), jnp.int32),
            pltpu.VMEM((_G, 5), jnp.float32),
            pltpu.VMEM((5, _NC), jnp.float32),
            pltpu.VMEM((5, _NC), jnp.int32),
            pltpu.VMEM((5, _NC), jnp.float32),
            pltpu.VMEM((5, _NC), jnp.int32),
            pltpu.VMEM((4400,), jnp.float32),
            pltpu.VMEM((16,), jnp.float32),
            pltpu.VMEM((16,), jnp.float32),
            pltpu.SemaphoreType.DMA,
            pltpu.SemaphoreType.DMA,
            pltpu.SemaphoreType.DMA,
            pltpu.SemaphoreType.DMA,
        ],
        compiler_params=pltpu.CompilerParams(needs_layout_passes=False),
    )(tgt, pcat)


def kernel(pred0, pred1, pred2, targets):
    pcat = jnp.concatenate(
        [pred0.reshape(-1), pred1.reshape(-1), pred2.reshape(-1)])
    part = _sc_loss(targets, pcat)
    o = jnp.sum(part, axis=0)
    soft, box, items, tpos, corr = o[0], o[1], o[2], o[3], o[4]
    return (soft - corr) / jnp.maximum(1.0, tpos) + 5.0 * box / jnp.maximum(1.0, items)
